# baseline (speedup 1.0000x reference)
import jax
import jax.numpy as jnp
from jax import lax
from jax.experimental import pallas as pl
from jax.experimental.pallas import tpu as pltpu
from jax.experimental.pallas import tpu_sc as plsc

_NCOMP = 16
_NNODES = 207
_NT = 12
_NPLANE = _NNODES * _NT
_DIM = _NCOMP * _NPLANE
_B = 1024
_NH = 24
_ND = 7
_NCOMBO = _NH * _ND
_LUTW = 168
_L = 16

_F_PER_W = 1248
_F_SUB = 208
_N_SUB = _F_PER_W // _F_SUB
_CBLK = _F_SUB * _LUTW

_P_PER_W = 78
_P_CHUNK = 13
_N_PCH = _P_PER_W // _P_CHUNK
_LUTBLK = _NCOMP * _P_CHUNK * _LUTW
_PLANE = _NCOMP * _B


def _worker_id():
  return lax.axis_index("s") * 2 + lax.axis_index("c")


def _phase1_body(hf, df, combo, hblk, dblk, cblk, sem_s, sem_o):
  wid = _worker_id()
  f0 = jnp.minimum(wid * _F_PER_W, _DIM - _F_PER_W)
  iota = lax.iota(jnp.int32, _L)

  pltpu.async_copy(combo.at[pl.ds(0, _CBLK)], cblk.at[pl.ds(0, _CBLK)], sem_o)
  pltpu.async_copy(combo.at[pl.ds(0, _CBLK)], cblk.at[pl.ds(_CBLK, _CBLK)],
                   sem_o)

  def subchunk(s, carry):
    fs = f0 + s * _F_SUB
    par = lax.rem(s, 2) * _CBLK
    pltpu.async_copy(hf.at[:, pl.ds(fs, _F_SUB)], hblk, sem_s)
    pltpu.async_copy(df.at[:, pl.ds(fs, _F_SUB)], dblk, sem_s)
    pltpu.make_async_copy(hf.at[:, pl.ds(0, _F_SUB)], hblk, sem_s).wait()
    pltpu.make_async_copy(df.at[:, pl.ds(0, _F_SUB)], dblk, sem_s).wait()
    pltpu.make_async_copy(combo.at[pl.ds(0, _CBLK)],
                          cblk.at[pl.ds(0, _CBLK)], sem_o).wait()

    @plsc.parallel_loop(0, _F_SUB // _L, unroll=2)
    def feat(s16):
      sl = pl.ds(s16 * _L, _L)
      fbase = par + (s16 * _L + iota) * _LUTW
      for j in range(_NCOMBO):
        v = hblk[j // _ND, sl] + dblk[j % _ND, sl]
        plsc.store_scatter(cblk, [fbase + j], v)
    pltpu.async_copy(cblk.at[pl.ds(par, _CBLK)],
                     combo.at[pl.ds(fs * _LUTW, _CBLK)], sem_o)
    return carry

  lax.fori_loop(0, _N_SUB, subchunk, 0)
  pltpu.make_async_copy(combo.at[pl.ds(0, _CBLK)],
                        cblk.at[pl.ds(0, _CBLK)], sem_o).wait()
  pltpu.make_async_copy(combo.at[pl.ds(0, _CBLK)],
                        cblk.at[pl.ds(0, _CBLK)], sem_o).wait()


def _phase2_body(combo, hh, dh, out, hs, ds_, cid, lutblk, pbuf, sem_l,
                 sem_p):
  wid = _worker_id()
  nt0 = jnp.minimum(wid * _P_PER_W, _NPLANE - _P_PER_W)

  pltpu.sync_copy(hh, hs)
  pltpu.sync_copy(dh, ds_)

  def mkcid(g, c2):
    sl = pl.ds(g * _L, _L)
    cid[sl] = hs[sl] * _ND + ds_[sl]
    return c2

  lax.fori_loop(0, _B // _L, mkcid, 0)

  for c in range(_NCOMP):
    pltpu.async_copy(
        combo.at[pl.ds((c * _NPLANE + nt0) * _LUTW, _P_CHUNK * _LUTW)],
        lutblk.at[pl.ds(c * _P_CHUNK * _LUTW, _P_CHUNK * _LUTW)], sem_l)
  pltpu.async_copy(out.at[pl.ds(0, _PLANE)], pbuf.at[pl.ds(0, _PLANE)],
                   sem_p)
  pltpu.async_copy(out.at[pl.ds(0, _PLANE)], pbuf.at[pl.ds(_PLANE, _PLANE)],
                   sem_p)

  def do_plane(k, dnt, lpar):
    ntg = nt0 + k * _P_CHUNK + dnt
    p = k * _P_CHUNK + dnt
    ppar = lax.rem(p, 2) * _PLANE
    lbases = [lpar + (c * _P_CHUNK + dnt) * _LUTW for c in range(_NCOMP)]
    pltpu.make_async_copy(out.at[pl.ds(0, _PLANE)],
                          pbuf.at[pl.ds(0, _PLANE)], sem_p).wait()

    @plsc.parallel_loop(0, _B // _L, unroll=2)
    def grp(g):
      cv = cid[pl.ds(g * _L, _L)]
      gh3 = (g // 8) * 1024
      gl4 = lax.rem(g, 8) * 16
      for c in range(_NCOMP):
        vals = plsc.load_gather(lutblk, [cv + lbases[c]])
        off = ppar + (c // 8) * 8192 + gh3 + (c % 8) * 128 + gl4
        pbuf[pl.ds(pl.multiple_of(off, 16), _L)] = vals
    pltpu.async_copy(pbuf.at[pl.ds(ppar, _PLANE)],
                     out.at[pl.ds(ntg * _PLANE, _PLANE)], sem_p)

  def chunk(k, carry):
    lpar = lax.rem(k, 2) * _LUTBLK
    nxt = jnp.minimum(nt0 + (k + 1) * _P_CHUNK, _NPLANE - _P_CHUNK)
    for c in range(_NCOMP):
      pltpu.async_copy(
          combo.at[pl.ds((c * _NPLANE + nxt) * _LUTW, _P_CHUNK * _LUTW)],
          lutblk.at[pl.ds((_LUTBLK - lpar) + c * _P_CHUNK * _LUTW,
                          _P_CHUNK * _LUTW)], sem_l)
    pltpu.make_async_copy(combo.at[pl.ds(0, _LUTBLK)],
                          lutblk.at[pl.ds(0, _LUTBLK)], sem_l).wait()

    def pair(i, c2):
      do_plane(k, i * 2, lpar)
      do_plane(k, i * 2 + 1, lpar)
      return c2

    lax.fori_loop(0, (_P_CHUNK - 1) // 2, pair, 0)
    do_plane(k, _P_CHUNK - 1, lpar)
    return carry

  lax.fori_loop(0, _N_PCH, chunk, 0)
  pltpu.make_async_copy(combo.at[pl.ds(0, _LUTBLK)],
                        lutblk.at[pl.ds(0, _LUTBLK)], sem_l).wait()
  pltpu.make_async_copy(out.at[pl.ds(0, _PLANE)],
                        pbuf.at[pl.ds(0, _PLANE)], sem_p).wait()
  pltpu.make_async_copy(out.at[pl.ds(0, _PLANE)],
                        pbuf.at[pl.ds(0, _PLANE)], sem_p).wait()


@jax.jit
def _sc_call(hf, df, hh, dh):
  mesh = plsc.VectorSubcoreMesh(core_axis_name="c", subcore_axis_name="s")
  params = pltpu.CompilerParams(use_tc_tiling_on_sc=False,
                                needs_layout_passes=False)
  combo = pl.kernel(
      _phase1_body,
      out_type=jax.ShapeDtypeStruct((_DIM * _LUTW,), jnp.float32),
      mesh=mesh,
      compiler_params=params,
      scratch_types=[
          pltpu.VMEM((_NH, _F_SUB), jnp.float32),
          pltpu.VMEM((_ND, _F_SUB), jnp.float32),
          pltpu.VMEM((2 * _CBLK,), jnp.float32),
          pltpu.SemaphoreType.DMA,
          pltpu.SemaphoreType.DMA,
      ],
  )(hf, df)
  return pl.kernel(
      _phase2_body,
      out_type=jax.ShapeDtypeStruct((_NPLANE * _PLANE,), jnp.float32),
      mesh=mesh,
      compiler_params=params,
      scratch_types=[
          pltpu.VMEM((_B,), jnp.int32),
          pltpu.VMEM((_B,), jnp.int32),
          pltpu.VMEM((_B,), jnp.int32),
          pltpu.VMEM((2 * _LUTBLK,), jnp.float32),
          pltpu.VMEM((2 * _PLANE,), jnp.float32),
          pltpu.SemaphoreType.DMA,
          pltpu.SemaphoreType.DMA,
      ],
  )(combo, hh, dh)


def kernel(H, D, h_ebd, d_ebd):
  out1d = _sc_call(h_ebd, d_ebd, H.astype(jnp.int32), D.astype(jnp.int32))
  out6 = out1d.reshape(_NNODES, _NT, 2, 8, 8, 128)
  return out6.transpose(3, 5, 2, 4, 0, 1).reshape(_B, _NCOMP, _NNODES, _NT)


# speedup vs baseline: 1.4788x; 1.4788x over previous
import jax
import jax.numpy as jnp
from jax import lax
from jax.experimental import pallas as pl
from jax.experimental.pallas import tpu as pltpu
from jax.experimental.pallas import tpu_sc as plsc

_NCOMP = 16
_NNODES = 207
_NT = 12
_NPLANE = _NNODES * _NT
_DIM = _NCOMP * _NPLANE
_B = 1024
_NH = 24
_ND = 7
_NCOMBO = _NH * _ND
_LUTW = 168
_L = 16

_F_PER_W = 1248
_F_SUB = 208
_N_SUB = _F_PER_W // _F_SUB
_CBLK = _F_SUB * _LUTW

_P_PER_W = 78
_P_CHUNK = 13
_N_PCH = _P_PER_W // _P_CHUNK
_LUTBLK = _NCOMP * _P_CHUNK * _LUTW
_PLANE = _NCOMP * _B


def _worker_id():
  return lax.axis_index("s") * 2 + lax.axis_index("c")


def _phase1_body(hf, df, combo, hblk, dblk, cblk, sem_s, sem_o):
  wid = _worker_id()
  f0 = jnp.minimum(wid * _F_PER_W, _DIM - _F_PER_W)
  iota = lax.iota(jnp.int32, _L)

  pltpu.async_copy(combo.at[pl.ds(0, _CBLK)], cblk.at[pl.ds(0, _CBLK)], sem_o)
  pltpu.async_copy(combo.at[pl.ds(0, _CBLK)], cblk.at[pl.ds(_CBLK, _CBLK)],
                   sem_o)

  def subchunk(s, carry):
    fs = f0 + s * _F_SUB
    par = lax.rem(s, 2) * _CBLK
    pltpu.async_copy(hf.at[:, pl.ds(fs, _F_SUB)], hblk, sem_s)
    pltpu.async_copy(df.at[:, pl.ds(fs, _F_SUB)], dblk, sem_s)
    pltpu.make_async_copy(hf.at[:, pl.ds(0, _F_SUB)], hblk, sem_s).wait()
    pltpu.make_async_copy(df.at[:, pl.ds(0, _F_SUB)], dblk, sem_s).wait()
    pltpu.make_async_copy(combo.at[pl.ds(0, _CBLK)],
                          cblk.at[pl.ds(0, _CBLK)], sem_o).wait()

    @plsc.parallel_loop(0, _F_SUB // _L, unroll=1)
    def feat(s16):
      sl = pl.ds(s16 * _L, _L)
      fbase = par + (s16 * _L + iota) * _LUTW
      for j in range(_NCOMBO):
        v = hblk[j // _ND, sl] + dblk[j % _ND, sl]
        plsc.store_scatter(cblk, [fbase + j], v)
    pltpu.async_copy(cblk.at[pl.ds(par, _CBLK)],
                     combo.at[pl.ds(fs * _LUTW, _CBLK)], sem_o)
    return carry

  lax.fori_loop(0, _N_SUB, subchunk, 0)
  pltpu.make_async_copy(combo.at[pl.ds(0, _CBLK)],
                        cblk.at[pl.ds(0, _CBLK)], sem_o).wait()
  pltpu.make_async_copy(combo.at[pl.ds(0, _CBLK)],
                        cblk.at[pl.ds(0, _CBLK)], sem_o).wait()


def _phase2_body(combo, hh, dh, out, hs, ds_, cid, lutblk, pbuf, sem_l,
                 sem_p):
  wid = _worker_id()
  nt0 = jnp.minimum(wid * _P_PER_W, _NPLANE - _P_PER_W)

  pltpu.sync_copy(hh, hs)
  pltpu.sync_copy(dh, ds_)

  def mkcid(g, c2):
    sl = pl.ds(g * _L, _L)
    cid[sl] = hs[sl] * _ND + ds_[sl]
    return c2

  lax.fori_loop(0, _B // _L, mkcid, 0)

  for c in range(_NCOMP):
    pltpu.async_copy(
        combo.at[pl.ds((c * _NPLANE + nt0) * _LUTW, _P_CHUNK * _LUTW)],
        lutblk.at[pl.ds(c * _P_CHUNK * _LUTW, _P_CHUNK * _LUTW)], sem_l)
  pltpu.async_copy(out.at[pl.ds(0, _PLANE)], pbuf.at[pl.ds(0, _PLANE)],
                   sem_p)
  pltpu.async_copy(out.at[pl.ds(0, _PLANE)], pbuf.at[pl.ds(_PLANE, _PLANE)],
                   sem_p)

  def do_plane(k, dnt, lpar):
    ntg = nt0 + k * _P_CHUNK + dnt
    p = k * _P_CHUNK + dnt
    ppar = lax.rem(p, 2) * _PLANE
    lbases = [lpar + (c * _P_CHUNK + dnt) * _LUTW for c in range(_NCOMP)]
    pltpu.make_async_copy(out.at[pl.ds(0, _PLANE)],
                          pbuf.at[pl.ds(0, _PLANE)], sem_p).wait()

    @plsc.parallel_loop(0, _B // _L, unroll=2)
    def grp(g):
      cv = cid[pl.ds(g * _L, _L)]
      gh3 = (g // 8) * 1024
      gl4 = lax.rem(g, 8) * 16
      for c in range(_NCOMP):
        vals = plsc.load_gather(lutblk, [cv + lbases[c]])
        off = ppar + (c // 8) * 8192 + gh3 + (c % 8) * 128 + gl4
        pbuf[pl.ds(pl.multiple_of(off, 16), _L)] = vals
    pltpu.async_copy(pbuf.at[pl.ds(ppar, _PLANE)],
                     out.at[pl.ds(ntg * _PLANE, _PLANE)], sem_p)

  def chunk(k, carry):
    lpar = lax.rem(k, 2) * _LUTBLK
    nxt = jnp.minimum(nt0 + (k + 1) * _P_CHUNK, _NPLANE - _P_CHUNK)
    for c in range(_NCOMP):
      pltpu.async_copy(
          combo.at[pl.ds((c * _NPLANE + nxt) * _LUTW, _P_CHUNK * _LUTW)],
          lutblk.at[pl.ds((_LUTBLK - lpar) + c * _P_CHUNK * _LUTW,
                          _P_CHUNK * _LUTW)], sem_l)
    pltpu.make_async_copy(combo.at[pl.ds(0, _LUTBLK)],
                          lutblk.at[pl.ds(0, _LUTBLK)], sem_l).wait()

    def pair(i, c2):
      do_plane(k, i * 2, lpar)
      do_plane(k, i * 2 + 1, lpar)
      return c2

    lax.fori_loop(0, (_P_CHUNK - 1) // 2, pair, 0)
    do_plane(k, _P_CHUNK - 1, lpar)
    return carry

  lax.fori_loop(0, _N_PCH, chunk, 0)
  pltpu.make_async_copy(combo.at[pl.ds(0, _LUTBLK)],
                        lutblk.at[pl.ds(0, _LUTBLK)], sem_l).wait()
  pltpu.make_async_copy(out.at[pl.ds(0, _PLANE)],
                        pbuf.at[pl.ds(0, _PLANE)], sem_p).wait()
  pltpu.make_async_copy(out.at[pl.ds(0, _PLANE)],
                        pbuf.at[pl.ds(0, _PLANE)], sem_p).wait()


@jax.jit
def _sc_call(hf, df, hh, dh):
  mesh = plsc.VectorSubcoreMesh(core_axis_name="c", subcore_axis_name="s")
  params = pltpu.CompilerParams(use_tc_tiling_on_sc=False,
                                needs_layout_passes=False)
  combo = pl.kernel(
      _phase1_body,
      out_type=jax.ShapeDtypeStruct((_DIM * _LUTW,), jnp.float32),
      mesh=mesh,
      compiler_params=params,
      scratch_types=[
          pltpu.VMEM((_NH, _F_SUB), jnp.float32),
          pltpu.VMEM((_ND, _F_SUB), jnp.float32),
          pltpu.VMEM((2 * _CBLK,), jnp.float32),
          pltpu.SemaphoreType.DMA,
          pltpu.SemaphoreType.DMA,
      ],
  )(hf, df)
  return pl.kernel(
      _phase2_body,
      out_type=jax.ShapeDtypeStruct((_NPLANE * _PLANE,), jnp.float32),
      mesh=mesh,
      compiler_params=params,
      scratch_types=[
          pltpu.VMEM((_B,), jnp.int32),
          pltpu.VMEM((_B,), jnp.int32),
          pltpu.VMEM((_B,), jnp.int32),
          pltpu.VMEM((2 * _LUTBLK,), jnp.float32),
          pltpu.VMEM((2 * _PLANE,), jnp.float32),
          pltpu.SemaphoreType.DMA,
          pltpu.SemaphoreType.DMA,
      ],
  )(combo, hh, dh)


def kernel(H, D, h_ebd, d_ebd):
  out1d = _sc_call(h_ebd, d_ebd, H.astype(jnp.int32), D.astype(jnp.int32))
  out6 = out1d.reshape(_NNODES, _NT, 2, 8, 8, 128)
  return out6.transpose(3, 5, 2, 4, 0, 1).reshape(_B, _NCOMP, _NNODES, _NT)
